# baseline (device time: 24714 ns/iter reference)
import jax
import jax.numpy as jnp
from jax import lax
from jax.experimental import pallas as pl
from jax.experimental.pallas import tpu as pltpu

N_DEV = 16
N_PLANE = 4
N_LOCAL_E = 4
ROWS = 1024
ROWS_PER_DEV = ROWS // N_DEV
BLK = N_PLANE * ROWS_PER_DEV
D_MODEL = 256
H = 512
N_EXPERTS = 64
KCAT = N_LOCAL_E * D_MODEL


def kernel(x, router_W, route_idx, expert_W):
    def body(
        x_ref,
        rw_ref,
        idx_ref,
        ew_ref,
        out_ref,
        xw_ref,
        ewcat_ref,
        partial_ref,
        comm1_ref,
        ps_ref,
        comm2_ref,
        ssem1,
        rsem1,
        ssem2,
        rsem2,
    ):
        my = lax.axis_index("i")
        z = lax.div(my, N_PLANE)
        r = lax.rem(my, N_PLANE)

        bsem = pltpu.get_barrier_semaphore()
        for dr in range(1, N_PLANE):
            peer = N_PLANE * z + lax.rem(r + dr, N_PLANE)
            pl.semaphore_signal(
                bsem, inc=1, device_id=(peer,),
                device_id_type=pl.DeviceIdType.MESH,
            )
        for dz in range(1, N_PLANE):
            peer = N_PLANE * lax.rem(z + dz, N_PLANE) + r
            pl.semaphore_signal(
                bsem, inc=1, device_id=(peer,),
                device_id_type=pl.DeviceIdType.MESH,
            )
        pl.semaphore_wait(bsem, 6)

        scores = jnp.dot(
            x_ref[:, :], rw_ref[:, :], preferred_element_type=jnp.float32
        )
        m = jnp.max(scores, axis=1, keepdims=True)
        p = jnp.exp(scores - m)
        p = p / jnp.sum(p, axis=1, keepdims=True)
        e0 = idx_ref[:, 0:1]
        e1 = idx_ref[:, 1:2]
        lanes = lax.broadcasted_iota(jnp.int32, (ROWS, N_EXPERTS), 1)
        g0 = jnp.sum(jnp.where(lanes == e0, p, 0.0), axis=1, keepdims=True)
        g1 = jnp.sum(jnp.where(lanes == e1, p, 0.0), axis=1, keepdims=True)
        gs = g0 + g1

        for l in range(N_LOCAL_E):
            e = my * N_LOCAL_E + l
            c = jnp.where(e0 == e, g0, 0.0) + jnp.where(e1 == e, g1, 0.0)
            xw_ref[:, l * D_MODEL:(l + 1) * D_MODEL] = (
                x_ref[:, :] * (c / gs)
            ).astype(jnp.bfloat16)
            ewcat_ref[l * D_MODEL:(l + 1) * D_MODEL, :] = ew_ref[l].astype(
                jnp.bfloat16
            )

        rdmas = []
        for zp in range(N_PLANE):
            block = jnp.dot(
                xw_ref[zp * BLK:(zp + 1) * BLK, :],
                ewcat_ref[:, :],
                preferred_element_type=jnp.float32,
            )
            partial_ref[zp * BLK:(zp + 1) * BLK, :] = block.astype(jnp.bfloat16)
            for dr in range(1, N_PLANE):
                g = lax.rem(r + dr, N_PLANE)
                rdma = pltpu.make_async_remote_copy(
                    src_ref=partial_ref.at[
                        pl.ds(zp * BLK + g * ROWS_PER_DEV, ROWS_PER_DEV), :
                    ],
                    dst_ref=comm1_ref.at[dr - 1, zp],
                    send_sem=ssem1.at[dr - 1, zp],
                    recv_sem=rsem1.at[dr - 1, zp],
                    device_id=(N_PLANE * z + g,),
                    device_id_type=pl.DeviceIdType.MESH,
                )
                rdma.start()
                rdmas.append(rdma)

        base = None
        rdmas2 = []
        for zi in range(N_PLANE):
            zpd = lax.rem(z + zi, N_PLANE)
            s = partial_ref[
                pl.ds(zpd * BLK + r * ROWS_PER_DEV, ROWS_PER_DEV), :
            ].astype(jnp.float32)
            for dr in range(1, N_PLANE):
                wait = pltpu.make_async_remote_copy(
                    src_ref=comm1_ref.at[dr - 1, zpd],
                    dst_ref=comm1_ref.at[dr - 1, zpd],
                    send_sem=ssem1.at[dr - 1, 0],
                    recv_sem=rsem1.at[dr - 1, zpd],
                    device_id=(my,),
                    device_id_type=pl.DeviceIdType.MESH,
                )
                wait.wait_recv()
                s = s + comm1_ref[dr - 1, zpd].astype(jnp.float32)
            if zi == 0:
                base = s
                continue
            ps_ref[zpd, :, :] = s.astype(jnp.bfloat16)
            rdma = pltpu.make_async_remote_copy(
                src_ref=ps_ref.at[zpd],
                dst_ref=comm2_ref.at[3 - zi],
                send_sem=ssem2.at[zi - 1],
                recv_sem=rsem2.at[3 - zi],
                device_id=(N_PLANE * zpd + r,),
                device_id_type=pl.DeviceIdType.MESH,
            )
            rdma.start()
            rdmas2.append(rdma)

        for slot in (2, 1, 0):
            wait = pltpu.make_async_remote_copy(
                src_ref=comm2_ref.at[slot],
                dst_ref=comm2_ref.at[slot],
                send_sem=ssem2.at[0],
                recv_sem=rsem2.at[slot],
                device_id=(my,),
                device_id_type=pl.DeviceIdType.MESH,
            )
            wait.wait_recv()
            base = base + comm2_ref[slot].astype(jnp.float32)
        out_ref[:, :] = base

        for rdma in rdmas:
            rdma.wait_send()
        for rdma in rdmas2:
            rdma.wait_send()

    return pl.pallas_call(
        body,
        out_shape=jax.ShapeDtypeStruct((ROWS_PER_DEV, H), jnp.float32),
        in_specs=[
            pl.BlockSpec(memory_space=pltpu.VMEM),
            pl.BlockSpec(memory_space=pltpu.VMEM),
            pl.BlockSpec(memory_space=pltpu.VMEM),
            pl.BlockSpec(memory_space=pltpu.VMEM),
        ],
        out_specs=pl.BlockSpec(memory_space=pltpu.VMEM),
        scratch_shapes=[
            pltpu.VMEM((ROWS, KCAT), jnp.bfloat16),
            pltpu.VMEM((KCAT, H), jnp.bfloat16),
            pltpu.VMEM((ROWS, H), jnp.bfloat16),
            pltpu.VMEM((3, N_PLANE, ROWS_PER_DEV, H), jnp.bfloat16),
            pltpu.VMEM((N_PLANE, ROWS_PER_DEV, H), jnp.bfloat16),
            pltpu.VMEM((3, ROWS_PER_DEV, H), jnp.bfloat16),
            pltpu.SemaphoreType.DMA((3, N_PLANE)),
            pltpu.SemaphoreType.DMA((3, N_PLANE)),
            pltpu.SemaphoreType.DMA((3,)),
            pltpu.SemaphoreType.DMA((3,)),
        ],
        compiler_params=pltpu.CompilerParams(collective_id=0),
    )(x, router_W, route_idx, expert_W)


# device time: 22095 ns/iter; 1.1185x vs baseline; 1.1185x over previous
import jax
import jax.numpy as jnp
from jax import lax
from jax.experimental import pallas as pl
from jax.experimental.pallas import tpu as pltpu

N_DEV = 16
N_LOCAL_E = 4
ROWS = 1024
ROWS_PER_DEV = ROWS // N_DEV
D_MODEL = 256
H = 512
N_EXPERTS = 64
BLK = 4 * ROWS_PER_DEV
PAD = BLK - ROWS_PER_DEV
KCAT = N_LOCAL_E * D_MODEL


def kernel(x, router_W, route_idx, expert_W):
    def body(
        x_ref,
        rw_ref,
        idx_ref,
        ew_ref,
        out_ref,
        xw_ref,
        ewcat_ref,
        stage0_ref,
        stage1_ref,
        stage2_ref,
        stage3_ref,
        comm_ref,
        send_sems,
        recv_sems,
    ):
        my = lax.axis_index("i")
        stage_refs = [stage0_ref, stage1_ref, stage2_ref, stage3_ref]

        bsem = pltpu.get_barrier_semaphore()
        for k in range(1, N_DEV):
            peer = lax.rem(my + k, N_DEV)
            pl.semaphore_signal(
                bsem, inc=1, device_id=(peer,),
                device_id_type=pl.DeviceIdType.MESH,
            )

        scores = jnp.dot(
            x_ref[:, :], rw_ref[:, :], preferred_element_type=jnp.float32
        )
        m = jnp.max(scores, axis=1, keepdims=True)
        p = jnp.exp(scores - m)
        p = p / jnp.sum(p, axis=1, keepdims=True)
        e0 = idx_ref[:, 0:1]
        e1 = idx_ref[:, 1:2]
        lanes = lax.broadcasted_iota(jnp.int32, (ROWS, N_EXPERTS), 1)
        g0 = jnp.sum(jnp.where(lanes == e0, p, 0.0), axis=1, keepdims=True)
        g1 = jnp.sum(jnp.where(lanes == e1, p, 0.0), axis=1, keepdims=True)
        gs = g0 + g1

        for l in range(N_LOCAL_E):
            e = my * N_LOCAL_E + l
            c = jnp.where(e0 == e, g0, 0.0) + jnp.where(e1 == e, g1, 0.0)
            xw_ref[:ROWS, l * D_MODEL:(l + 1) * D_MODEL] = (
                x_ref[:, :] * (c / gs)
            ).astype(jnp.bfloat16)
            ewcat_ref[l * D_MODEL:(l + 1) * D_MODEL, :] = ew_ref[l].astype(
                jnp.bfloat16
            )
        xw_ref[ROWS:, :] = xw_ref[:PAD, :]

        pl.semaphore_wait(bsem, N_DEV - 1)

        rdmas = []
        oacc = None
        for j in range(4):
            r0 = lax.rem(my + (4 * j + 1), N_DEV) * ROWS_PER_DEV
            block = jnp.dot(
                xw_ref[pl.ds(r0, BLK), :],
                ewcat_ref[:, :],
                preferred_element_type=jnp.float32,
            )
            stage_refs[j][:, :] = block.astype(jnp.bfloat16)
            for c in range(4):
                k = 4 * j + 1 + c
                if k == N_DEV:
                    oacc = block[PAD:, :]
                    continue
                rdma = pltpu.make_async_remote_copy(
                    src_ref=stage_refs[j].at[
                        pl.ds(c * ROWS_PER_DEV, ROWS_PER_DEV), :
                    ],
                    dst_ref=comm_ref.at[k],
                    send_sem=send_sems.at[k],
                    recv_sem=recv_sems.at[k],
                    device_id=(lax.rem(my + k, N_DEV),),
                    device_id_type=pl.DeviceIdType.MESH,
                )
                rdma.start()
                rdmas.append(rdma)

        for k in range(1, N_DEV):
            rdmas[k - 1].wait_recv()
            oacc = oacc + comm_ref[k].astype(jnp.float32)
        out_ref[:, :] = oacc

        for k in range(1, N_DEV):
            rdmas[k - 1].wait_send()

    return pl.pallas_call(
        body,
        out_shape=jax.ShapeDtypeStruct((ROWS_PER_DEV, H), jnp.float32),
        in_specs=[
            pl.BlockSpec(memory_space=pltpu.VMEM),
            pl.BlockSpec(memory_space=pltpu.VMEM),
            pl.BlockSpec(memory_space=pltpu.VMEM),
            pl.BlockSpec(memory_space=pltpu.VMEM),
        ],
        out_specs=pl.BlockSpec(memory_space=pltpu.VMEM),
        scratch_shapes=[
            pltpu.VMEM((ROWS + PAD, KCAT), jnp.bfloat16),
            pltpu.VMEM((KCAT, H), jnp.bfloat16),
            pltpu.VMEM((BLK, H), jnp.bfloat16),
            pltpu.VMEM((BLK, H), jnp.bfloat16),
            pltpu.VMEM((BLK, H), jnp.bfloat16),
            pltpu.VMEM((BLK, H), jnp.bfloat16),
            pltpu.VMEM((N_DEV, ROWS_PER_DEV, H), jnp.bfloat16),
            pltpu.SemaphoreType.DMA((N_DEV,)),
            pltpu.SemaphoreType.DMA((N_DEV,)),
        ],
        compiler_params=pltpu.CompilerParams(collective_id=0),
    )(x, router_W, route_idx, expert_W)
